# Initial kernel scaffold; baseline (speedup 1.0000x reference)
import functools

import jax
import jax.numpy as jnp
from jax import lax
from jax.experimental import pallas as pl
from jax.experimental.pallas import tpu as pltpu
from jax.experimental.pallas import tpu_sc as plsc

N = 25000
E = 312500
NC, NS = 2, 16
NW = NC * NS
CH = 128
CPW = 77
E_PAD = NW * CPW * CH
NV_PAD = 25008
NA_PAD = 25088
RPS = NA_PAD // NS
BN = 3136


def _make_segsum(D):
    mesh = plsc.VectorSubcoreMesh(core_axis_name="c", subcore_axis_name="s")

    @functools.partial(
        pl.kernel,
        mesh=mesh,
        out_type=jax.ShapeDtypeStruct((NC * NA_PAD, D), jnp.float32),
        scratch_types=[
            pltpu.VMEM((CPW, CH), jnp.int32),
            pltpu.VMEM((CPW, CH), jnp.int32),
            pltpu.VMEM((CH, D), jnp.float32),
            pltpu.VMEM_SHARED((NA_PAD, D), jnp.float32),
            pltpu.SemaphoreType.DMA,
        ],
    )
    def seg(values_h, src_h, dst_h, zeros_h, out_h, src_v, dst_v, rows_v, acc_s, sem):
        cid = lax.axis_index("c")
        sid = lax.axis_index("s")
        wid = sid * NC + cid

        pltpu.sync_copy(src_h.at[pl.ds(wid * CPW, CPW)], src_v)
        pltpu.sync_copy(dst_h.at[pl.ds(wid * CPW, CPW)], dst_v)
        pltpu.sync_copy(zeros_h.at[pl.ds(sid * RPS, RPS)],
                        acc_s.at[pl.ds(sid * RPS, RPS)])
        plsc.subcore_barrier()

        def body(j, carry):
            pltpu.async_copy(values_h.at[src_v.at[j]], rows_v, sem).wait()
            pltpu.sync_copy(rows_v, acc_s.at[dst_v.at[j]], add=True)
            return carry

        lax.fori_loop(0, CPW, body, 0, unroll=False)
        plsc.subcore_barrier()

        pltpu.sync_copy(acc_s.at[pl.ds(sid * RPS, RPS)],
                        out_h.at[pl.ds(cid * NA_PAD + sid * RPS, RPS)])

    return seg


_segsum8 = _make_segsum(8)
_segsum16 = _make_segsum(16)


def _stage_b(s2, xi, WB):
    def body(s2_r, xi_r, wb_r, o_r):
        s = s2_r[0] + s2_r[1]
        acc = jnp.broadcast_to(wb_r[9:10, :], (BN, 16))
        for c in range(4):
            acc = acc + s[:, c:c + 1] * wb_r[c:c + 1, :]
        acc = acc + s[:, 4:5] * wb_r[4:5, :]
        for c in range(4):
            acc = acc + xi_r[:, c:c + 1] * wb_r[5 + c:6 + c, :]
        o_r[...] = acc

    return pl.pallas_call(
        body,
        grid=(NA_PAD // BN,),
        in_specs=[
            pl.BlockSpec((2, BN, 8), lambda i: (0, i, 0)),
            pl.BlockSpec((BN, 4), lambda i: (i, 0)),
            pl.BlockSpec((16, 16), lambda i: (0, 0)),
        ],
        out_specs=pl.BlockSpec((BN, 16), lambda i: (i, 0)),
        out_shape=jax.ShapeDtypeStruct((NA_PAD, 16), jnp.float32),
    )(s2, xi, WB)


def _stage_d(tt2, xu, WD):
    def body(t2_r, xu_r, wd_r, o_r):
        o = t2_r[0] + t2_r[1]
        acc = o + jnp.broadcast_to(wd_r[9:10, :], (BN, 16))
        for c in range(4):
            acc = acc + o[:, 10 + c:11 + c] * wd_r[c:c + 1, :]
        acc = acc + o[:, 14:15] * wd_r[4:5, :]
        for c in range(4):
            acc = acc + xu_r[:, c:c + 1] * wd_r[5 + c:6 + c, :]
        o_r[...] = acc

    return pl.pallas_call(
        body,
        grid=(NA_PAD // BN,),
        in_specs=[
            pl.BlockSpec((2, BN, 16), lambda i: (0, i, 0)),
            pl.BlockSpec((BN, 4), lambda i: (i, 0)),
            pl.BlockSpec((16, 16), lambda i: (0, 0)),
        ],
        out_specs=pl.BlockSpec((BN, 16), lambda i: (i, 0)),
        out_shape=jax.ShapeDtypeStruct((NA_PAD, 16), jnp.float32),
    )(tt2, xu, WD)


def _prep_edges(ei):
    npad = E_PAD - E
    src = jnp.concatenate([ei[0], jnp.full((npad,), N, jnp.int32)])
    dst = jnp.concatenate([ei[1], jnp.full((npad,), N, jnp.int32)])
    return src.reshape(-1, CH), dst.reshape(-1, CH)


def _pad_rows(x, rows):
    return jnp.pad(x, ((0, rows - x.shape[0]), (0, 0)))


def _fold(We, be, Wmat):
    W3 = Wmat.reshape(4, 32, -1)
    return jnp.einsum("ck,ckj->cj", We, W3), jnp.einsum("ck,ckj->j", be, W3)


def _pad16(a):
    a = jnp.atleast_2d(a)
    return jnp.pad(a, ((0, 0), (0, 16 - a.shape[1])))


def kernel(x_user, x_item, edge_u2i, edge_i2u,
           emb_W_user, emb_b_user, emb_W_item, emb_b_item,
           Wl1_u, bl1_u, Wr1_u, Wl1_i, bl1_i, Wr1_i,
           Wl2_u, bl2_u, Wr2_u, Wl2_i, bl2_i, Wr2_i,
           Wm, bm):
    C = Wl2_u @ Wm
    G = Wr2_u @ Wm
    e = bl2_u @ Wm + bm
    Ml_i, cl_i = _fold(emb_W_user, emb_b_user, Wl1_i)
    Mr_i, cr_i = _fold(emb_W_item, emb_b_item, Wr1_i)
    P, p, Rm, r0 = Ml_i @ C, cl_i @ C, Mr_i @ C, (bl1_i + cr_i) @ C
    Ml_u, cl_u = _fold(emb_W_item, emb_b_item, Wl1_u)
    Mr_u, cr_u = _fold(emb_W_user, emb_b_user, Wr1_u)
    P2, p2, R2, r2 = Ml_u @ G, cl_u @ G, Mr_u @ G, (bl1_u + cr_u) @ G + e

    RS = _pad16(Rm).at[jnp.arange(4), 10 + jnp.arange(4)].set(1.0)
    cB = _pad16(r0).at[0, 14].set(1.0)
    WB = jnp.concatenate([_pad16(P), _pad16(p), RS, cB,
                          jnp.zeros((6, 16), jnp.float32)], axis=0)
    WD = jnp.concatenate([_pad16(P2), _pad16(p2), _pad16(R2), _pad16(r2),
                          jnp.zeros((6, 16), jnp.float32)], axis=0)

    v1 = jnp.zeros((NV_PAD, 8), jnp.float32)
    v1 = v1.at[:N, :4].set(x_user).at[:N, 4].set(1.0)
    srcA, dstA = _prep_edges(edge_u2i)
    z8 = jnp.zeros((NA_PAD, 8), jnp.float32)
    sA = _segsum8(v1, srcA, dstA, z8).reshape(NC, NA_PAD, 8)

    xi_p = _pad_rows(x_item, NA_PAD)
    qq = _stage_b(sA, xi_p, WB)

    srcC, dstC = _prep_edges(edge_i2u)
    z16 = jnp.zeros((NA_PAD, 16), jnp.float32)
    tt = _segsum16(qq, srcC, dstC, z16).reshape(NC, NA_PAD, 16)

    xu_p = _pad_rows(x_user, NA_PAD)
    out16 = _stage_d(tt, xu_p, WD)
    return out16[:N, :10]


# speedup vs baseline: 12.8547x; 12.8547x over previous
import functools

import jax
import jax.numpy as jnp
from jax import lax
from jax.experimental import pallas as pl
from jax.experimental.pallas import tpu as pltpu
from jax.experimental.pallas import tpu_sc as plsc

N = 25000
E = 312500
NC, NS = 2, 16
NW = NC * NS
CH = 128
CPW = 80
E_PAD = NW * CPW * CH
NV_PAD = 25008
NA_PAD = 25088
RPS = NA_PAD // NS
BN = 3136


def _make_segsum(D):
    mesh = plsc.VectorSubcoreMesh(core_axis_name="c", subcore_axis_name="s")

    @functools.partial(
        pl.kernel,
        mesh=mesh,
        compiler_params=pltpu.CompilerParams(use_tc_tiling_on_sc=False),
        out_type=jax.ShapeDtypeStruct((NC * NA_PAD, D), jnp.float32),
        scratch_types=[
            pltpu.VMEM((CPW, CH), jnp.int32),
            pltpu.VMEM((CPW, CH), jnp.int32),
            pltpu.VMEM((CH, D), jnp.float32),
            pltpu.VMEM_SHARED((NA_PAD, D), jnp.float32),
            pltpu.SemaphoreType.DMA,
        ],
    )
    def seg(values_h, src_h, dst_h, zeros_h, out_h, src_v, dst_v, rows_v, acc_s, sem):
        cid = lax.axis_index("c")
        sid = lax.axis_index("s")
        wid = sid * NC + cid

        pltpu.sync_copy(src_h.at[pl.ds(wid * CPW, CPW)], src_v)
        pltpu.sync_copy(dst_h.at[pl.ds(wid * CPW, CPW)], dst_v)
        pltpu.sync_copy(zeros_h.at[pl.ds(sid * RPS, RPS)],
                        acc_s.at[pl.ds(sid * RPS, RPS)])
        plsc.subcore_barrier()

        def body(j, carry):
            pltpu.async_copy(values_h.at[src_v.at[j]], rows_v, sem).wait()
            pltpu.sync_copy(rows_v, acc_s.at[dst_v.at[j]], add=True)
            return carry

        lax.fori_loop(0, CPW, body, 0, unroll=False)
        plsc.subcore_barrier()

        pltpu.sync_copy(acc_s.at[pl.ds(sid * RPS, RPS)],
                        out_h.at[pl.ds(cid * NA_PAD + sid * RPS, RPS)])

    return seg


_segsum8 = _make_segsum(8)
_segsum16 = _make_segsum(16)


def _stage_b(s2, xi, WB):
    def body(s2_r, xi_r, wb_r, o_r):
        s = s2_r[0] + s2_r[1]
        acc = jnp.broadcast_to(wb_r[9:10, :], (BN, 16))
        for c in range(4):
            acc = acc + s[:, c:c + 1] * wb_r[c:c + 1, :]
        acc = acc + s[:, 4:5] * wb_r[4:5, :]
        for c in range(4):
            acc = acc + xi_r[:, c:c + 1] * wb_r[5 + c:6 + c, :]
        o_r[...] = acc

    return pl.pallas_call(
        body,
        grid=(NA_PAD // BN,),
        in_specs=[
            pl.BlockSpec((2, BN, 8), lambda i: (0, i, 0)),
            pl.BlockSpec((BN, 4), lambda i: (i, 0)),
            pl.BlockSpec((16, 16), lambda i: (0, 0)),
        ],
        out_specs=pl.BlockSpec((BN, 16), lambda i: (i, 0)),
        out_shape=jax.ShapeDtypeStruct((NA_PAD, 16), jnp.float32),
    )(s2, xi, WB)


def _stage_d(tt2, xu, WD):
    def body(t2_r, xu_r, wd_r, o_r):
        o = t2_r[0] + t2_r[1]
        acc = o + jnp.broadcast_to(wd_r[9:10, :], (BN, 16))
        for c in range(4):
            acc = acc + o[:, 10 + c:11 + c] * wd_r[c:c + 1, :]
        acc = acc + o[:, 14:15] * wd_r[4:5, :]
        for c in range(4):
            acc = acc + xu_r[:, c:c + 1] * wd_r[5 + c:6 + c, :]
        o_r[...] = acc

    return pl.pallas_call(
        body,
        grid=(NA_PAD // BN,),
        in_specs=[
            pl.BlockSpec((2, BN, 16), lambda i: (0, i, 0)),
            pl.BlockSpec((BN, 4), lambda i: (i, 0)),
            pl.BlockSpec((16, 16), lambda i: (0, 0)),
        ],
        out_specs=pl.BlockSpec((BN, 16), lambda i: (i, 0)),
        out_shape=jax.ShapeDtypeStruct((NA_PAD, 16), jnp.float32),
    )(tt2, xu, WD)


def _prep_edges(ei):
    npad = E_PAD - E
    src = jnp.concatenate([ei[0], jnp.full((npad,), N, jnp.int32)])
    dst = jnp.concatenate(
        [ei[1], N + (jnp.arange(npad, dtype=jnp.int32) % (NA_PAD - N))])
    return src.reshape(-1, CH), dst.reshape(-1, CH)


def _pad_rows(x, rows):
    return jnp.pad(x, ((0, rows - x.shape[0]), (0, 0)))


_HI = jax.lax.Precision.HIGHEST


def _mm(a, b):
    return jnp.matmul(a, b, precision=_HI)


def _fold(We, be, Wmat):
    W3 = Wmat.reshape(4, 32, -1)
    return (jnp.einsum("ck,ckj->cj", We, W3, precision=_HI),
            jnp.einsum("ck,ckj->j", be, W3, precision=_HI))


def _pad16(a):
    a = jnp.atleast_2d(a)
    return jnp.pad(a, ((0, 0), (0, 16 - a.shape[1])))


def kernel(x_user, x_item, edge_u2i, edge_i2u,
           emb_W_user, emb_b_user, emb_W_item, emb_b_item,
           Wl1_u, bl1_u, Wr1_u, Wl1_i, bl1_i, Wr1_i,
           Wl2_u, bl2_u, Wr2_u, Wl2_i, bl2_i, Wr2_i,
           Wm, bm):
    C = _mm(Wl2_u, Wm)
    G = _mm(Wr2_u, Wm)
    e = _mm(bl2_u, Wm) + bm
    Ml_i, cl_i = _fold(emb_W_user, emb_b_user, Wl1_i)
    Mr_i, cr_i = _fold(emb_W_item, emb_b_item, Wr1_i)
    P, p, Rm, r0 = _mm(Ml_i, C), _mm(cl_i, C), _mm(Mr_i, C), _mm(bl1_i + cr_i, C)
    Ml_u, cl_u = _fold(emb_W_item, emb_b_item, Wl1_u)
    Mr_u, cr_u = _fold(emb_W_user, emb_b_user, Wr1_u)
    P2, p2, R2, r2 = (_mm(Ml_u, G), _mm(cl_u, G), _mm(Mr_u, G),
                      _mm(bl1_u + cr_u, G) + e)

    RS = _pad16(Rm).at[jnp.arange(4), 10 + jnp.arange(4)].set(1.0)
    cB = _pad16(r0).at[0, 14].set(1.0)
    WB = jnp.concatenate([_pad16(P), _pad16(p), RS, cB,
                          jnp.zeros((6, 16), jnp.float32)], axis=0)
    WD = jnp.concatenate([_pad16(P2), _pad16(p2), _pad16(R2), _pad16(r2),
                          jnp.zeros((6, 16), jnp.float32)], axis=0)

    v1 = jnp.zeros((NV_PAD, 8), jnp.float32)
    v1 = v1.at[:N, :4].set(x_user).at[:N, 4].set(1.0)
    srcA, dstA = _prep_edges(edge_u2i)
    z8 = jnp.zeros((NA_PAD, 8), jnp.float32)
    sA = _segsum8(v1, srcA, dstA, z8).reshape(NC, NA_PAD, 8)

    xi_p = _pad_rows(x_item, NA_PAD)
    qq = _stage_b(sA, xi_p, WB)

    srcC, dstC = _prep_edges(edge_i2u)
    z16 = jnp.zeros((NA_PAD, 16), jnp.float32)
    tt = _segsum16(qq, srcC, dstC, z16).reshape(NC, NA_PAD, 16)

    xu_p = _pad_rows(x_user, NA_PAD)
    out16 = _stage_d(tt, xu_p, WD)
    return out16[:N, :10]


# speedup vs baseline: 16.9826x; 1.3211x over previous
import functools

import jax
import jax.numpy as jnp
from jax import lax
from jax.experimental import pallas as pl
from jax.experimental.pallas import tpu as pltpu
from jax.experimental.pallas import tpu_sc as plsc

N = 25000
E = 312500
NC, NS = 2, 16
NW = NC * NS
CH = 128
CPW = 80
E_PAD = NW * CPW * CH
NA_PAD = 25088
RPS = NA_PAD // NS
BN = 5000


def _make_segsum(D):
    mesh = plsc.VectorSubcoreMesh(core_axis_name="c", subcore_axis_name="s")

    @functools.partial(
        pl.kernel,
        mesh=mesh,
        compiler_params=pltpu.CompilerParams(use_tc_tiling_on_sc=False),
        out_type=jax.ShapeDtypeStruct((NC, NA_PAD, D), jnp.float32),
        scratch_types=[
            pltpu.VMEM((CPW, CH), jnp.int32),
            pltpu.VMEM((CPW, CH), jnp.int32),
            pltpu.VMEM((CH, D), jnp.float32),
            pltpu.VMEM((CH, D), jnp.float32),
            pltpu.VMEM_SHARED((NA_PAD, D), jnp.float32),
            pltpu.SemaphoreType.DMA,
            pltpu.SemaphoreType.DMA,
        ],
    )
    def seg(values_h, src_h, dst_h, zeros_h, out_h,
            src_v, dst_v, rows_a, rows_b, acc_s, sem_a, sem_b):
        cid = lax.axis_index("c")
        sid = lax.axis_index("s")
        wid = sid * NC + cid

        pltpu.sync_copy(src_h.at[pl.ds(wid * CPW, CPW)], src_v)
        pltpu.sync_copy(dst_h.at[pl.ds(wid * CPW, CPW)], dst_v)
        pltpu.sync_copy(zeros_h.at[pl.ds(sid * RPS, RPS)],
                        acc_s.at[pl.ds(sid * RPS, RPS)])
        plsc.subcore_barrier()

        pltpu.async_copy(values_h.at[src_v.at[0]], rows_a, sem_a)

        def body(i, carry):
            j0 = i * 2
            pltpu.async_copy(values_h.at[src_v.at[j0 + 1]], rows_b, sem_b)
            pltpu.make_async_copy(values_h.at[src_v.at[j0]], rows_a, sem_a).wait()
            pltpu.sync_copy(rows_a, acc_s.at[dst_v.at[j0]], add=True)

            @pl.when(j0 + 2 < CPW)
            def _():
                pltpu.async_copy(values_h.at[src_v.at[j0 + 2]], rows_a, sem_a)

            pltpu.make_async_copy(values_h.at[src_v.at[j0 + 1]], rows_b, sem_b).wait()
            pltpu.sync_copy(rows_b, acc_s.at[dst_v.at[j0 + 1]], add=True)
            return carry

        lax.fori_loop(0, CPW // 2, body, 0, unroll=False)
        plsc.subcore_barrier()

        pltpu.sync_copy(acc_s.at[pl.ds(sid * RPS, RPS)],
                        out_h.at[cid, pl.ds(sid * RPS, RPS)])

    return seg


_segsum8 = _make_segsum(8)
_segsum16 = _make_segsum(16)


def _pack8(xu):
    def body(x_r, o_r):
        o_r[...] = jnp.concatenate(
            [x_r[...],
             jnp.ones((BN, 1), jnp.float32),
             jnp.zeros((BN, 3), jnp.float32)], axis=1)

    return pl.pallas_call(
        body,
        grid=(N // BN,),
        in_specs=[pl.BlockSpec((BN, 4), lambda i: (i, 0))],
        out_specs=pl.BlockSpec((BN, 8), lambda i: (i, 0)),
        out_shape=jax.ShapeDtypeStruct((N, 8), jnp.float32),
    )(xu)


def _stage_b(s2, xi, WB):
    def body(s2_r, xi_r, wb_r, o_r):
        s = s2_r[0] + s2_r[1]
        acc = jnp.broadcast_to(wb_r[9:10, :], (BN, 16))
        for c in range(4):
            acc = acc + s[:, c:c + 1] * wb_r[c:c + 1, :]
        acc = acc + s[:, 4:5] * wb_r[4:5, :]
        for c in range(4):
            acc = acc + xi_r[:, c:c + 1] * wb_r[5 + c:6 + c, :]
        o_r[...] = acc

    return pl.pallas_call(
        body,
        grid=(N // BN,),
        in_specs=[
            pl.BlockSpec((2, BN, 8), lambda i: (0, i, 0)),
            pl.BlockSpec((BN, 4), lambda i: (i, 0)),
            pl.BlockSpec((16, 16), lambda i: (0, 0)),
        ],
        out_specs=pl.BlockSpec((BN, 16), lambda i: (i, 0)),
        out_shape=jax.ShapeDtypeStruct((N, 16), jnp.float32),
    )(s2, xi, WB)


def _stage_d(tt2, xu, WD):
    def body(t2_r, xu_r, wd_r, o_r):
        o = t2_r[0] + t2_r[1]
        acc = o + jnp.broadcast_to(wd_r[9:10, :], (BN, 16))
        for c in range(4):
            acc = acc + o[:, 10 + c:11 + c] * wd_r[c:c + 1, :]
        acc = acc + o[:, 14:15] * wd_r[4:5, :]
        for c in range(4):
            acc = acc + xu_r[:, c:c + 1] * wd_r[5 + c:6 + c, :]
        o_r[...] = acc[:, :10]

    return pl.pallas_call(
        body,
        grid=(N // BN,),
        in_specs=[
            pl.BlockSpec((2, BN, 16), lambda i: (0, i, 0)),
            pl.BlockSpec((BN, 4), lambda i: (i, 0)),
            pl.BlockSpec((16, 16), lambda i: (0, 0)),
        ],
        out_specs=pl.BlockSpec((BN, 10), lambda i: (i, 0)),
        out_shape=jax.ShapeDtypeStruct((N, 10), jnp.float32),
    )(tt2, xu, WD)


def _prep_edges(ei):
    npad = E_PAD - E
    src = jnp.concatenate([ei[0], jnp.zeros((npad,), jnp.int32)])
    dst = jnp.concatenate(
        [ei[1], N + (jnp.arange(npad, dtype=jnp.int32) % (NA_PAD - N))])
    return src.reshape(-1, CH), dst.reshape(-1, CH)


_HI = jax.lax.Precision.HIGHEST


def _mm(a, b):
    return jnp.matmul(a, b, precision=_HI)


def _fold(We, be, Wmat):
    W3 = Wmat.reshape(4, 32, -1)
    return (jnp.einsum("ck,ckj->cj", We, W3, precision=_HI),
            jnp.einsum("ck,ckj->j", be, W3, precision=_HI))


def _pad16(a):
    a = jnp.atleast_2d(a)
    return jnp.pad(a, ((0, 0), (0, 16 - a.shape[1])))


def kernel(x_user, x_item, edge_u2i, edge_i2u,
           emb_W_user, emb_b_user, emb_W_item, emb_b_item,
           Wl1_u, bl1_u, Wr1_u, Wl1_i, bl1_i, Wr1_i,
           Wl2_u, bl2_u, Wr2_u, Wl2_i, bl2_i, Wr2_i,
           Wm, bm):
    C = _mm(Wl2_u, Wm)
    G = _mm(Wr2_u, Wm)
    e = _mm(bl2_u, Wm) + bm
    Ml_i, cl_i = _fold(emb_W_user, emb_b_user, Wl1_i)
    Mr_i, cr_i = _fold(emb_W_item, emb_b_item, Wr1_i)
    P, p, Rm, r0 = _mm(Ml_i, C), _mm(cl_i, C), _mm(Mr_i, C), _mm(bl1_i + cr_i, C)
    Ml_u, cl_u = _fold(emb_W_item, emb_b_item, Wl1_u)
    Mr_u, cr_u = _fold(emb_W_user, emb_b_user, Wr1_u)
    P2, p2, R2, r2 = (_mm(Ml_u, G), _mm(cl_u, G), _mm(Mr_u, G),
                      _mm(bl1_u + cr_u, G) + e)

    RS = _pad16(Rm).at[jnp.arange(4), 10 + jnp.arange(4)].set(1.0)
    cB = _pad16(r0).at[0, 14].set(1.0)
    WB = jnp.concatenate([_pad16(P), _pad16(p), RS, cB,
                          jnp.zeros((6, 16), jnp.float32)], axis=0)
    WD = jnp.concatenate([_pad16(P2), _pad16(p2), _pad16(R2), _pad16(r2),
                          jnp.zeros((6, 16), jnp.float32)], axis=0)

    v1 = _pack8(x_user)
    srcA, dstA = _prep_edges(edge_u2i)
    z8 = jnp.zeros((NA_PAD, 8), jnp.float32)
    sA = _segsum8(v1, srcA, dstA, z8)

    qq = _stage_b(sA, x_item, WB)

    srcC, dstC = _prep_edges(edge_i2u)
    z16 = jnp.zeros((NA_PAD, 16), jnp.float32)
    tt = _segsum16(qq, srcC, dstC, z16)

    return _stage_d(tt, x_user, WD)


# speedup vs baseline: 20.0792x; 1.1823x over previous
import functools

import jax
import jax.numpy as jnp
from jax import lax
from jax.experimental import pallas as pl
from jax.experimental.pallas import tpu as pltpu
from jax.experimental.pallas import tpu_sc as plsc

N = 25000
E = 312500
NC, NS = 2, 16
NW = NC * NS
CH = 128
CPW = 80
EPW = CPW * CH
E_PAD = NW * EPW
NA_PAD = 25088
RPS = NA_PAD // NS
NBUF = 4


def _make_segsum(D):
    mesh = plsc.VectorSubcoreMesh(core_axis_name="c", subcore_axis_name="s")

    @functools.partial(
        pl.kernel,
        mesh=mesh,
        compiler_params=pltpu.CompilerParams(use_tc_tiling_on_sc=False),
        out_type=jax.ShapeDtypeStruct((NC, NA_PAD, D), jnp.float32),
        scratch_types=(
            [pltpu.VMEM((EPW,), jnp.int32),
             pltpu.VMEM((EPW,), jnp.int32)]
            + [pltpu.VMEM((CH, D), jnp.float32) for _ in range(NBUF)]
            + [pltpu.VMEM_SHARED((NA_PAD, D), jnp.float32)]
            + [pltpu.SemaphoreType.DMA for _ in range(2 * NBUF)]
        ),
    )
    def seg(values_h, src_h, dst_h, zeros_h, out_h, src_v, dst_v, *rest):
        rows = rest[:NBUF]
        acc_s = rest[NBUF]
        gsem = rest[NBUF + 1:NBUF + 1 + NBUF]
        ssem = rest[NBUF + 1 + NBUF:]
        cid = lax.axis_index("c")
        sid = lax.axis_index("s")
        wid = sid * NC + cid
        base = wid * EPW

        pltpu.sync_copy(src_h.at[pl.ds(base, EPW)], src_v)
        pltpu.sync_copy(dst_h.at[pl.ds(base, EPW)], dst_v)
        pltpu.sync_copy(zeros_h.at[pl.ds(sid * RPS, RPS)],
                        acc_s.at[pl.ds(sid * RPS, RPS)])
        plsc.subcore_barrier()

        def gather(jj, b):
            pltpu.async_copy(values_h.at[src_v.at[pl.ds(jj * CH, CH)]],
                             rows[b], gsem[b])

        def gather_wait(jj, b):
            pltpu.make_async_copy(values_h.at[src_v.at[pl.ds(jj * CH, CH)]],
                                  rows[b], gsem[b]).wait()

        def scatter(jj, b):
            pltpu.async_copy(rows[b], acc_s.at[dst_v.at[pl.ds(jj * CH, CH)]],
                             ssem[b], add=True)

        def scatter_wait(jj, b):
            pltpu.make_async_copy(rows[b],
                                  acc_s.at[dst_v.at[pl.ds(jj * CH, CH)]],
                                  ssem[b]).wait()

        for b in range(NBUF):
            gather(b, b)

        def body(i, carry):
            j0 = i * NBUF
            for b in range(NBUF):
                jj = j0 + b
                gather_wait(jj, b)
                scatter(jj, b)
                pb = (b + NBUF - 1) % NBUF
                pjj = jj - 1 + NBUF

                @pl.when(jnp.logical_and(jj >= 1, pjj < CPW))
                def _():
                    scatter_wait(pjj - NBUF, pb)
                    gather(pjj, pb)
            return carry

        lax.fori_loop(0, CPW // NBUF, body, 0, unroll=False)
        for b in range(NBUF):
            scatter_wait(CPW - NBUF + b, b)
        plsc.subcore_barrier()

        pltpu.sync_copy(acc_s.at[pl.ds(sid * RPS, RPS)],
                        out_h.at[cid, pl.ds(sid * RPS, RPS)])

    return seg


_SEG = {}


def _segsum(D, *args):
    if D not in _SEG:
        _SEG[D] = _make_segsum(D)
    return _SEG[D](*args)


_HI = jax.lax.Precision.HIGHEST


def _stage_b(s2p, xip, A, B, bias):
    def body(s_r, x_r, a_r, b_r, c_r, o_r):
        s = s_r[0] + s_r[1]
        o_r[...] = (jnp.dot(s, a_r[...], precision=_HI)
                    + jnp.dot(x_r[...], b_r[...], precision=_HI)
                    + c_r[...])

    full = lambda shape: pl.BlockSpec(shape, lambda: tuple(0 for _ in shape))
    return pl.pallas_call(
        body,
        in_specs=[full((2, 1568, 128)), full((1568, 64)),
                  full((128, 256)), full((64, 256)), full((1, 256))],
        out_specs=full((1568, 256)),
        out_shape=jax.ShapeDtypeStruct((1568, 256), jnp.float32),
    )(s2p, xip, A, B, bias)


def _stage_d(ttp, xup, A, B, bias):
    def body(t_r, x_r, a_r, b_r, c_r, o_r):
        o = t_r[0] + t_r[1]
        o_r[...] = (jnp.dot(o, a_r[...], precision=_HI)
                    + jnp.dot(x_r[...], b_r[...], precision=_HI)
                    + c_r[...])

    full = lambda shape: pl.BlockSpec(shape, lambda: tuple(0 for _ in shape))
    return pl.pallas_call(
        body,
        in_specs=[full((2, 3136, 128)), full((3136, 32)),
                  full((128, 128)), full((32, 128)), full((1, 128))],
        out_specs=full((3136, 128)),
        out_shape=jax.ShapeDtypeStruct((3136, 128), jnp.float32),
    )(ttp, xup, A, B, bias)


def _prep_edges(ei):
    npad = E_PAD - E
    src = jnp.concatenate([ei[0], jnp.zeros((npad,), jnp.int32)])
    dst = jnp.concatenate(
        [ei[1], N + (jnp.arange(npad, dtype=jnp.int32) % (NA_PAD - N))])
    return src, dst


def _fold(We, be, Wmat):
    W3 = Wmat.reshape(4, 32, -1)
    return (jnp.einsum("ck,ckj->cj", We, W3, precision=_HI),
            jnp.einsum("ck,ckj->j", be, W3, precision=_HI))


def _mm(a, b):
    return jnp.matmul(a, b, precision=_HI)


def _pad16(a):
    a = jnp.atleast_2d(a)
    return jnp.pad(a, ((0, 0), (0, 16 - a.shape[1])))


def _blockdiag(E_node, groups):
    k = E_node.shape[0]
    eye = jnp.eye(groups, dtype=jnp.float32)
    return (eye[:, None, :, None] * E_node[None, :, None, :]).reshape(
        groups * k, groups * 16)


def _pack_cols(x, per_row):
    xp = jnp.pad(x, ((0, NA_PAD - N), (0, 0)))
    return xp.reshape(-1, per_row)


def kernel(x_user, x_item, edge_u2i, edge_i2u,
           emb_W_user, emb_b_user, emb_W_item, emb_b_item,
           Wl1_u, bl1_u, Wr1_u, Wl1_i, bl1_i, Wr1_i,
           Wl2_u, bl2_u, Wr2_u, Wl2_i, bl2_i, Wr2_i,
           Wm, bm):
    C = _mm(Wl2_u, Wm)
    G = _mm(Wr2_u, Wm)
    e = _mm(bl2_u, Wm) + bm
    Ml_i, cl_i = _fold(emb_W_user, emb_b_user, Wl1_i)
    Mr_i, cr_i = _fold(emb_W_item, emb_b_item, Wr1_i)
    P, p, Rm, r0 = _mm(Ml_i, C), _mm(cl_i, C), _mm(Mr_i, C), _mm(bl1_i + cr_i, C)
    Ml_u, cl_u = _fold(emb_W_item, emb_b_item, Wl1_u)
    Mr_u, cr_u = _fold(emb_W_user, emb_b_user, Wr1_u)
    P2, p2, R2, r2 = (_mm(Ml_u, G), _mm(cl_u, G), _mm(Mr_u, G),
                      _mm(bl1_u + cr_u, G) + e)

    E8 = jnp.concatenate([_pad16(P), _pad16(p), jnp.zeros((3, 16))], axis=0)
    RS = _pad16(Rm).at[jnp.arange(4), 10 + jnp.arange(4)].set(1.0)
    cB = _pad16(r0).at[0, 14].set(1.0)
    A_B = _blockdiag(E8, 16)
    B_B = _blockdiag(RS, 16)
    bias_B = jnp.tile(cB, (1, 16))

    E16 = jnp.eye(16, dtype=jnp.float32)
    E16 = E16.at[10:14, :].add(_pad16(P2))
    E16 = E16.at[14:15, :].add(_pad16(p2))
    cD = _pad16(r2)
    A_D = _blockdiag(E16, 8)
    B_D = _blockdiag(_pad16(R2), 8)
    bias_D = jnp.tile(cD, (1, 8))

    v1 = jnp.concatenate(
        [x_user, jnp.ones((N, 1), jnp.float32), jnp.zeros((N, 3), jnp.float32)],
        axis=1)
    srcA, dstA = _prep_edges(edge_u2i)
    z8 = jnp.zeros((NA_PAD, 8), jnp.float32)
    sA = _segsum(8, v1, srcA, dstA, z8)

    qq = _stage_b(sA.reshape(2, 1568, 128), _pack_cols(x_item, 64),
                  A_B, B_B, bias_B).reshape(NA_PAD, 16)

    srcC, dstC = _prep_edges(edge_i2u)
    z16 = jnp.zeros((NA_PAD, 16), jnp.float32)
    tt = _segsum(16, qq, srcC, dstC, z16)

    out16 = _stage_d(tt.reshape(2, 3136, 128), _pack_cols(x_user, 32),
                     A_D, B_D, bias_D).reshape(NA_PAD, 16)
    return out16[:N, :10]
